# baseline (speedup 1.0000x reference)
import functools

import jax
import jax.numpy as jnp
from jax import lax
from jax.experimental import pallas as pl
from jax.experimental.pallas import tpu as pltpu
from jax.experimental.pallas import tpu_sc as plsc

VOCAB = 100000
EMBED_DIM = 300
BATCH = 1024
EMBED_MAX_NORM = 1.0

TV = 2048


CHUNK = 128
N_CHUNKS = VOCAB * EMBED_DIM // CHUNK
K_CH = 4
L = 16


def _sc_gather(idx, emb_table):
    info = plsc.get_sparse_core_info()
    nc, ns = info.num_cores, info.num_subcores
    nw = nc * ns
    b_per_w = BATCH // nw
    ng = b_per_w // L

    tab128 = emb_table.reshape(N_CHUNKS, CHUNK)
    mesh = plsc.VectorSubcoreMesh(core_axis_name="c", subcore_axis_name="s")

    @functools.partial(
        pl.kernel,
        mesh=mesh,
        out_type=jax.ShapeDtypeStruct((BATCH, EMBED_DIM), jnp.float32),
        scratch_types=[
            pltpu.VMEM((b_per_w,), jnp.int32),
            pltpu.VMEM((b_per_w,), jnp.int32),
            pltpu.VMEM((K_CH, ng, L, CHUNK), jnp.float32),
            pltpu.VMEM((b_per_w, EMBED_DIM), jnp.float32),
            pltpu.SemaphoreType.DMA,
        ],
        compiler_params=pltpu.CompilerParams(use_tc_tiling_on_sc=False,
                                             needs_layout_passes=False),
    )
    def k(idx_hbm, table_hbm, out_hbm, idx_v, r_v, chunks_v, xout_v, sem):
        wid = lax.axis_index("s") * nc + lax.axis_index("c")
        base = wid * b_per_w
        pltpu.sync_copy(idx_hbm.at[pl.ds(base, b_per_w)], idx_v)

        lanes = lax.iota(jnp.int32, L)
        copies = []
        for g in range(ng):
            iv = idx_v[pl.ds(g * L, L)]
            w0 = iv * EMBED_DIM
            c0 = lax.shift_right_logical(w0, 7)
            r_v[pl.ds(g * L, L)] = w0 & 127
            for kk in range(K_CH):
                cvec = jnp.minimum(c0 + kk, N_CHUNKS - 1)
                copies.append(pltpu.async_copy(
                    table_hbm.at[cvec], chunks_v.at[kk, g], sem))
        for c in copies:
            c.wait()

        def body(d, carry):
            dv = jnp.full((L,), d, jnp.int32)
            for g in range(ng):
                rv = r_v[pl.ds(g * L, L)]
                p = rv + d
                kvec = lax.shift_right_logical(p, 7)
                jvec = p & 127
                gv = jnp.full((L,), g, jnp.int32)
                vals = plsc.load_gather(chunks_v, [kvec, gv, lanes, jvec])
                plsc.store_scatter(xout_v, [gv * L + lanes, dv], vals)
            return carry

        lax.fori_loop(0, EMBED_DIM, body, 0)
        pltpu.sync_copy(xout_v, out_hbm.at[pl.ds(base, b_per_w)])

    return k(idx, tab128)


def _proj_body(x_ref, w_ref, b_ref, o_ref, xs_ref):
    j = pl.program_id(0)

    @pl.when(j == 0)
    def _():
        x = x_ref[...]
        norm = jnp.sqrt(jnp.sum(x * x, axis=1, keepdims=True))
        scale = jnp.where(norm > EMBED_MAX_NORM,
                          EMBED_MAX_NORM / jnp.maximum(norm, 1e-7), 1.0)
        xs_ref[...] = x * scale

    acc = lax.dot_general(xs_ref[...], w_ref[...],
                          (((1,), (1,)), ((), ())),
                          preferred_element_type=jnp.float32)
    o_ref[...] = acc + b_ref[...]


def _tc_project(x, w, b2):
    nv = pl.cdiv(VOCAB, TV)
    return pl.pallas_call(
        _proj_body,
        grid=(nv,),
        in_specs=[
            pl.BlockSpec((BATCH, EMBED_DIM), lambda j: (0, 0)),
            pl.BlockSpec((TV, EMBED_DIM), lambda j: (j, 0)),
            pl.BlockSpec((1, TV), lambda j: (0, j)),
        ],
        out_specs=pl.BlockSpec((BATCH, TV), lambda j: (0, j)),
        out_shape=jax.ShapeDtypeStruct((BATCH, VOCAB), jnp.float32),
        scratch_shapes=[pltpu.VMEM((BATCH, EMBED_DIM), jnp.float32)],
    )(x, w, b2)


def kernel(inputs, emb_table, W, b):
    idx = inputs.astype(jnp.int32)
    x = _sc_gather(idx, emb_table)
    return _tc_project(x, W, b.reshape(1, VOCAB))


# speedup vs baseline: 2.2066x; 2.2066x over previous
import functools

import jax
import jax.numpy as jnp
from jax import lax
from jax.experimental import pallas as pl
from jax.experimental.pallas import tpu as pltpu
from jax.experimental.pallas import tpu_sc as plsc

VOCAB = 100000
EMBED_DIM = 300
BATCH = 1024
EMBED_MAX_NORM = 1.0

TV = 2048
L = 16
LANE_BLK = 128


def _sc_gather(idx, tabT):
    info = plsc.get_sparse_core_info()
    nc, ns = info.num_cores, info.num_subcores
    nw = nc * ns
    b_per_w = BATCH // nw

    mesh = plsc.VectorSubcoreMesh(core_axis_name="c", subcore_axis_name="s")

    @functools.partial(
        pl.kernel,
        mesh=mesh,
        out_type=jax.ShapeDtypeStruct((BATCH, EMBED_DIM), jnp.float32),
        scratch_types=[
            pltpu.VMEM((b_per_w,), jnp.int32),
            pltpu.VMEM((2, EMBED_DIM, LANE_BLK), jnp.float32),
            pltpu.VMEM((b_per_w, EMBED_DIM), jnp.float32),
            pltpu.SemaphoreType.DMA,
            pltpu.SemaphoreType.DMA,
        ],
        compiler_params=pltpu.CompilerParams(needs_layout_passes=False),
    )
    def k(idx_hbm, tab_hbm, out_hbm, idx_v, buf_v, xout_v, sem0, sem1):
        wid = lax.axis_index("s") * nc + lax.axis_index("c")
        base = wid * b_per_w
        pltpu.sync_copy(idx_hbm.at[pl.ds(base, b_per_w)], idx_v)

        lanes = lax.iota(jnp.int32, L)
        sems = (sem0, sem1)

        def row_scalar(i):
            g, r = divmod(i, L)
            iv = idx_v[pl.ds(g * L, L)]
            return lax.reduce_sum(jnp.where(lanes == r, iv, 0), axes=(0,))

        def start_fetch(i, s):
            off = pl.multiple_of(lax.shift_right_logical(s, 7) * LANE_BLK,
                                 LANE_BLK)
            return pltpu.async_copy(
                tab_hbm.at[:, pl.ds(off, LANE_BLK)],
                buf_v.at[i % 2], sems[i % 2])

        def extract(i, s):
            lv = jnp.full((L,), s & (LANE_BLK - 1), jnp.int32)
            iv16 = jnp.full((L,), i, jnp.int32)
            for t in range(EMBED_DIM // L + 1):
                dvec = jnp.full((L,), t * L, jnp.int32) + lanes
                mask = dvec < EMBED_DIM if (t + 1) * L > EMBED_DIM else None
                vals = plsc.load_gather(buf_v.at[i % 2], [dvec, lv],
                                        mask=mask)
                plsc.store_scatter(xout_v, [iv16, dvec], vals, mask=mask)

        scalars = [row_scalar(i) for i in range(b_per_w)]
        cp = start_fetch(0, scalars[0])
        for i in range(b_per_w):
            nxt = start_fetch(i + 1, scalars[i + 1]) if i + 1 < b_per_w else None
            cp.wait()
            extract(i, scalars[i])
            cp = nxt

        pltpu.sync_copy(xout_v, out_hbm.at[pl.ds(base, b_per_w)])

    return k(idx, tabT)


def _proj_body(x_ref, w_ref, b_ref, o_ref, xs_ref):
    j = pl.program_id(0)

    @pl.when(j == 0)
    def _():
        x = x_ref[...]
        norm = jnp.sqrt(jnp.sum(x * x, axis=1, keepdims=True))
        scale = jnp.where(norm > EMBED_MAX_NORM,
                          EMBED_MAX_NORM / jnp.maximum(norm, 1e-7), 1.0)
        xs_ref[...] = x * scale

    acc = lax.dot_general(xs_ref[...], w_ref[...],
                          (((1,), (0,)), ((), ())),
                          preferred_element_type=jnp.float32)
    o_ref[...] = acc + b_ref[...]


def _tc_project(x, wt, b2):
    nv = pl.cdiv(VOCAB, TV)
    return pl.pallas_call(
        _proj_body,
        grid=(nv,),
        in_specs=[
            pl.BlockSpec((BATCH, EMBED_DIM), lambda j: (0, 0)),
            pl.BlockSpec((EMBED_DIM, TV), lambda j: (0, j)),
            pl.BlockSpec((1, TV), lambda j: (0, j)),
        ],
        out_specs=pl.BlockSpec((BATCH, TV), lambda j: (0, j)),
        out_shape=jax.ShapeDtypeStruct((BATCH, VOCAB), jnp.float32),
        scratch_shapes=[pltpu.VMEM((BATCH, EMBED_DIM), jnp.float32)],
    )(x, wt, b2)


def kernel(inputs, emb_table, W, b):
    idx = inputs.astype(jnp.int32)
    x = _sc_gather(idx, jnp.transpose(emb_table))
    return _tc_project(x, jnp.transpose(W), b.reshape(1, VOCAB))


# speedup vs baseline: 4.7090x; 2.1341x over previous
import functools

import jax
import jax.numpy as jnp
from jax import lax
from jax.experimental import pallas as pl
from jax.experimental.pallas import tpu as pltpu
from jax.experimental.pallas import tpu_sc as plsc

VOCAB = 100000
EMBED_DIM = 300
BATCH = 1024
EMBED_MAX_NORM = 1.0

TV = 2048
L = 16
LANE_BLK = 128


def _sc_gather(idx, tabT):
    info = plsc.get_sparse_core_info()
    nc, ns = info.num_cores, info.num_subcores
    nw = nc * ns
    b_per_w = BATCH // nw

    mesh = plsc.VectorSubcoreMesh(core_axis_name="c", subcore_axis_name="s")

    @functools.partial(
        pl.kernel,
        mesh=mesh,
        out_type=jax.ShapeDtypeStruct((BATCH, EMBED_DIM), jnp.float32),
        scratch_types=[
            pltpu.VMEM((b_per_w,), jnp.int32),
            pltpu.VMEM((2, EMBED_DIM, LANE_BLK), jnp.float32),
            pltpu.VMEM((b_per_w, EMBED_DIM), jnp.float32),
            pltpu.SemaphoreType.DMA,
            pltpu.SemaphoreType.DMA,
        ],
        compiler_params=pltpu.CompilerParams(needs_layout_passes=False),
    )
    def k(idx_hbm, tab_hbm, out_hbm, idx_v, buf_v, xout_v, sem0, sem1):
        wid = lax.axis_index("s") * nc + lax.axis_index("c")
        base = wid * b_per_w
        pltpu.sync_copy(idx_hbm.at[pl.ds(base, b_per_w)], idx_v)

        lanes = lax.iota(jnp.int32, L)
        sems = (sem0, sem1)

        def row_scalar(i):
            g, r = divmod(i, L)
            iv = idx_v[pl.ds(g * L, L)]
            return lax.reduce_sum(jnp.where(lanes == r, iv, 0), axes=(0,))

        def start_fetch(i, s):
            off = pl.multiple_of(lax.shift_right_logical(s, 7) * LANE_BLK,
                                 LANE_BLK)
            return pltpu.async_copy(
                tab_hbm.at[:, pl.ds(off, LANE_BLK)],
                buf_v.at[i % 2], sems[i % 2])

        def extract(i, s):
            lv = jnp.full((L,), s & (LANE_BLK - 1), jnp.int32)
            iv16 = jnp.full((L,), i, jnp.int32)
            for t in range(EMBED_DIM // L + 1):
                dvec = jnp.full((L,), t * L, jnp.int32) + lanes
                mask = dvec < EMBED_DIM if (t + 1) * L > EMBED_DIM else None
                vals = plsc.load_gather(buf_v.at[i % 2], [dvec, lv],
                                        mask=mask)
                plsc.store_scatter(xout_v, [iv16, dvec], vals, mask=mask)

        scalars = [row_scalar(i) for i in range(b_per_w)]
        cp = start_fetch(0, scalars[0])
        for i in range(b_per_w):
            nxt = start_fetch(i + 1, scalars[i + 1]) if i + 1 < b_per_w else None
            cp.wait()
            extract(i, scalars[i])
            cp = nxt

        pltpu.sync_copy(xout_v, out_hbm.at[pl.ds(base, b_per_w)])

    return k(idx, tabT)


def _proj_body(x_ref, w_ref, b_ref, o_ref, xs_ref):
    j = pl.program_id(0)

    @pl.when(j == 0)
    def _():
        x = x_ref[...]
        norm = jnp.sqrt(jnp.sum(x * x, axis=1, keepdims=True))
        scale = jnp.where(norm > EMBED_MAX_NORM,
                          EMBED_MAX_NORM / jnp.maximum(norm, 1e-7), 1.0)
        xs_ref[...] = x * scale

    acc = lax.dot_general(w_ref[...], xs_ref[...],
                          (((0,), (1,)), ((), ())),
                          preferred_element_type=jnp.float32)
    o_ref[...] = acc + b_ref[...]


def _tc_project(x, wt, bcol):
    nv = pl.cdiv(VOCAB, TV)
    return pl.pallas_call(
        _proj_body,
        grid=(nv,),
        in_specs=[
            pl.BlockSpec((BATCH, EMBED_DIM), lambda j: (0, 0)),
            pl.BlockSpec((EMBED_DIM, TV), lambda j: (0, j)),
            pl.BlockSpec((TV, 1), lambda j: (j, 0)),
        ],
        out_specs=pl.BlockSpec((TV, BATCH), lambda j: (j, 0)),
        out_shape=jax.ShapeDtypeStruct((VOCAB, BATCH), jnp.float32),
        scratch_shapes=[pltpu.VMEM((BATCH, EMBED_DIM), jnp.float32)],
    )(x, wt, bcol)


def kernel(inputs, emb_table, W, b):
    idx = inputs.astype(jnp.int32)
    x = _sc_gather(idx, jnp.transpose(emb_table))
    outT = _tc_project(x, jnp.transpose(W), b.reshape(VOCAB, 1))
    return jnp.transpose(outT)


# speedup vs baseline: 4.7775x; 1.0145x over previous
import functools

import jax
import jax.numpy as jnp
from jax import lax
from jax.experimental import pallas as pl
from jax.experimental.pallas import tpu as pltpu
from jax.experimental.pallas import tpu_sc as plsc

VOCAB = 100000
EMBED_DIM = 300
BATCH = 1024
EMBED_MAX_NORM = 1.0

TV = 3072
L = 16
LANE_BLK = 128


def _sc_gather(idx, tabT):
    info = plsc.get_sparse_core_info()
    nc, ns = info.num_cores, info.num_subcores
    nw = nc * ns
    b_per_w = BATCH // nw

    mesh = plsc.VectorSubcoreMesh(core_axis_name="c", subcore_axis_name="s")

    @functools.partial(
        pl.kernel,
        mesh=mesh,
        out_type=jax.ShapeDtypeStruct((BATCH, EMBED_DIM), jnp.float32),
        scratch_types=[
            pltpu.VMEM((b_per_w,), jnp.int32),
            pltpu.VMEM((2, EMBED_DIM, LANE_BLK), jnp.float32),
            pltpu.VMEM((b_per_w, EMBED_DIM), jnp.float32),
            pltpu.SemaphoreType.DMA,
            pltpu.SemaphoreType.DMA,
        ],
        compiler_params=pltpu.CompilerParams(needs_layout_passes=False),
    )
    def k(idx_hbm, tab_hbm, out_hbm, idx_v, buf_v, xout_v, sem0, sem1):
        wid = lax.axis_index("s") * nc + lax.axis_index("c")
        base = wid * b_per_w
        pltpu.sync_copy(idx_hbm.at[pl.ds(base, b_per_w)], idx_v)

        lanes = lax.iota(jnp.int32, L)
        sems = (sem0, sem1)

        def row_scalar(i):
            g, r = divmod(i, L)
            iv = idx_v[pl.ds(g * L, L)]
            return lax.reduce_sum(jnp.where(lanes == r, iv, 0), axes=(0,))

        def start_fetch(i, s):
            off = pl.multiple_of(lax.shift_right_logical(s, 7) * LANE_BLK,
                                 LANE_BLK)
            return pltpu.async_copy(
                tab_hbm.at[:, pl.ds(off, LANE_BLK)],
                buf_v.at[i % 2], sems[i % 2])

        def extract(i, s):
            lv = jnp.full((L,), s & (LANE_BLK - 1), jnp.int32)
            iv16 = jnp.full((L,), i, jnp.int32)
            for t in range(EMBED_DIM // L + 1):
                dvec = jnp.full((L,), t * L, jnp.int32) + lanes
                mask = dvec < EMBED_DIM if (t + 1) * L > EMBED_DIM else None
                vals = plsc.load_gather(buf_v.at[i % 2], [dvec, lv],
                                        mask=mask)
                plsc.store_scatter(xout_v, [iv16, dvec], vals, mask=mask)

        scalars = [row_scalar(i) for i in range(b_per_w)]
        cp = start_fetch(0, scalars[0])
        for i in range(b_per_w):
            nxt = start_fetch(i + 1, scalars[i + 1]) if i + 1 < b_per_w else None
            cp.wait()
            extract(i, scalars[i])
            cp = nxt

        pltpu.sync_copy(xout_v, out_hbm.at[pl.ds(base, b_per_w)])

    return k(idx, tabT)


def _proj_body(x_ref, w_ref, b_ref, o_ref, xs_ref):
    j = pl.program_id(0)

    @pl.when(j == 0)
    def _():
        x = x_ref[...]
        norm = jnp.sqrt(jnp.sum(x * x, axis=1, keepdims=True))
        scale = jnp.where(norm > EMBED_MAX_NORM,
                          EMBED_MAX_NORM / jnp.maximum(norm, 1e-7), 1.0)
        xs_ref[...] = x * scale

    acc = lax.dot_general(w_ref[...], xs_ref[...],
                          (((0,), (1,)), ((), ())),
                          preferred_element_type=jnp.float32)
    o_ref[...] = acc + b_ref[...]


def _tc_project(x, wt, bcol):
    nv = pl.cdiv(VOCAB, TV)
    return pl.pallas_call(
        _proj_body,
        grid=(nv,),
        in_specs=[
            pl.BlockSpec((BATCH, EMBED_DIM), lambda j: (0, 0)),
            pl.BlockSpec((EMBED_DIM, TV), lambda j: (0, j)),
            pl.BlockSpec((TV, 1), lambda j: (j, 0)),
        ],
        out_specs=pl.BlockSpec((TV, BATCH), lambda j: (j, 0)),
        out_shape=jax.ShapeDtypeStruct((VOCAB, BATCH), jnp.float32),
        scratch_shapes=[pltpu.VMEM((BATCH, EMBED_DIM), jnp.float32)],
    )(x, wt, bcol)


def kernel(inputs, emb_table, W, b):
    idx = inputs.astype(jnp.int32)
    x = _sc_gather(idx, jnp.transpose(emb_table))
    outT = _tc_project(x, jnp.transpose(W), b.reshape(VOCAB, 1))
    return jnp.transpose(outT)


# speedup vs baseline: 4.7996x; 1.0046x over previous
import functools

import jax
import jax.numpy as jnp
from jax import lax
from jax.experimental import pallas as pl
from jax.experimental.pallas import tpu as pltpu
from jax.experimental.pallas import tpu_sc as plsc

VOCAB = 100000
EMBED_DIM = 300
BATCH = 1024
EMBED_MAX_NORM = 1.0

TV = 3584
L = 16
LANE_BLK = 128


def _sc_gather(idx, tabT):
    info = plsc.get_sparse_core_info()
    nc, ns = info.num_cores, info.num_subcores
    nw = nc * ns
    b_per_w = BATCH // nw

    mesh = plsc.VectorSubcoreMesh(core_axis_name="c", subcore_axis_name="s")

    @functools.partial(
        pl.kernel,
        mesh=mesh,
        out_type=jax.ShapeDtypeStruct((BATCH, EMBED_DIM), jnp.float32),
        scratch_types=[
            pltpu.VMEM((b_per_w,), jnp.int32),
            pltpu.VMEM((2, EMBED_DIM, LANE_BLK), jnp.float32),
            pltpu.VMEM((b_per_w, EMBED_DIM), jnp.float32),
            pltpu.SemaphoreType.DMA,
            pltpu.SemaphoreType.DMA,
        ],
        compiler_params=pltpu.CompilerParams(needs_layout_passes=False),
    )
    def k(idx_hbm, tab_hbm, out_hbm, idx_v, buf_v, xout_v, sem0, sem1):
        wid = lax.axis_index("s") * nc + lax.axis_index("c")
        base = wid * b_per_w
        pltpu.sync_copy(idx_hbm.at[pl.ds(base, b_per_w)], idx_v)

        lanes = lax.iota(jnp.int32, L)
        sems = (sem0, sem1)

        def row_scalar(i):
            g, r = divmod(i, L)
            iv = idx_v[pl.ds(g * L, L)]
            return lax.reduce_sum(jnp.where(lanes == r, iv, 0), axes=(0,))

        def start_fetch(i, s):
            off = pl.multiple_of(lax.shift_right_logical(s, 7) * LANE_BLK,
                                 LANE_BLK)
            return pltpu.async_copy(
                tab_hbm.at[:, pl.ds(off, LANE_BLK)],
                buf_v.at[i % 2], sems[i % 2])

        def extract(i, s):
            lv = jnp.full((L,), s & (LANE_BLK - 1), jnp.int32)
            iv16 = jnp.full((L,), i, jnp.int32)
            for t in range(EMBED_DIM // L + 1):
                dvec = jnp.full((L,), t * L, jnp.int32) + lanes
                mask = dvec < EMBED_DIM if (t + 1) * L > EMBED_DIM else None
                vals = plsc.load_gather(buf_v.at[i % 2], [dvec, lv],
                                        mask=mask)
                plsc.store_scatter(xout_v, [iv16, dvec], vals, mask=mask)

        scalars = [row_scalar(i) for i in range(b_per_w)]
        cp = start_fetch(0, scalars[0])
        for i in range(b_per_w):
            nxt = start_fetch(i + 1, scalars[i + 1]) if i + 1 < b_per_w else None
            cp.wait()
            extract(i, scalars[i])
            cp = nxt

        pltpu.sync_copy(xout_v, out_hbm.at[pl.ds(base, b_per_w)])

    return k(idx, tabT)


def _proj_body(x_ref, w_ref, b_ref, o_ref, xs_ref):
    j = pl.program_id(0)

    @pl.when(j == 0)
    def _():
        x = x_ref[...]
        norm = jnp.sqrt(jnp.sum(x * x, axis=1, keepdims=True))
        scale = jnp.where(norm > EMBED_MAX_NORM,
                          EMBED_MAX_NORM / jnp.maximum(norm, 1e-7), 1.0)
        xs_ref[...] = x * scale

    acc = lax.dot_general(w_ref[...], xs_ref[...],
                          (((0,), (1,)), ((), ())),
                          preferred_element_type=jnp.float32)
    o_ref[...] = acc + b_ref[...]


def _tc_project(x, wt, bcol):
    nv = pl.cdiv(VOCAB, TV)
    return pl.pallas_call(
        _proj_body,
        grid=(nv,),
        in_specs=[
            pl.BlockSpec((BATCH, EMBED_DIM), lambda j: (0, 0)),
            pl.BlockSpec((EMBED_DIM, TV), lambda j: (0, j)),
            pl.BlockSpec((TV, 1), lambda j: (j, 0)),
        ],
        out_specs=pl.BlockSpec((TV, BATCH), lambda j: (j, 0)),
        out_shape=jax.ShapeDtypeStruct((VOCAB, BATCH), jnp.float32),
        scratch_shapes=[pltpu.VMEM((BATCH, EMBED_DIM), jnp.float32)],
    )(x, wt, bcol)


def kernel(inputs, emb_table, W, b):
    idx = inputs.astype(jnp.int32)
    x = _sc_gather(idx, jnp.transpose(emb_table))
    outT = _tc_project(x, jnp.transpose(W), b.reshape(VOCAB, 1))
    return jnp.transpose(outT)
